# baseline (speedup 1.0000x reference)
import functools

import jax
import jax.numpy as jnp
from jax import lax
from jax.experimental import pallas as pl
from jax.experimental.pallas import tpu as pltpu, tpu_sc as plsc

N_USERS = 30000
N_ITEMS = 20000
N_NODES = N_USERS + N_ITEMS
N_LAYERS = 2
ALPHA = 0.4
FEAT_DIM = 64


_EMBED_BLK = 1000


def _embed_body(img_ref, txt_ref, wi_ref, wt_ref, bias_ref, out0_ref, out1_ref):
    ie = jnp.dot(img_ref[...].astype(jnp.bfloat16), wi_ref[...].astype(jnp.bfloat16),
                 preferred_element_type=jnp.float32) + bias_ref[0, :FEAT_DIM]
    te = jnp.dot(txt_ref[...].astype(jnp.bfloat16), wt_ref[...].astype(jnp.bfloat16),
                 preferred_element_type=jnp.float32) + bias_ref[0, FEAT_DIM:]
    ni = jnp.sqrt(jnp.sum(ie * ie, axis=1, keepdims=True))
    nt = jnp.sqrt(jnp.sum(te * te, axis=1, keepdims=True))
    out0_ref[...] = ie / jnp.maximum(ni, 1e-12)
    out1_ref[...] = te / jnp.maximum(nt, 1e-12)


def _dense_embed(image_feats, text_feats, W_img, b_img, W_txt, b_txt):
    bias = jnp.concatenate([b_img, b_txt]).reshape(1, 2 * FEAT_DIM)
    return pl.pallas_call(
        _embed_body,
        grid=(N_ITEMS // _EMBED_BLK,),
        in_specs=[
            pl.BlockSpec((_EMBED_BLK, image_feats.shape[1]), lambda i: (i, 0)),
            pl.BlockSpec((_EMBED_BLK, text_feats.shape[1]), lambda i: (i, 0)),
            pl.BlockSpec(W_img.shape, lambda i: (0, 0)),
            pl.BlockSpec(W_txt.shape, lambda i: (0, 0)),
            pl.BlockSpec((1, 2 * FEAT_DIM), lambda i: (0, 0)),
        ],
        out_specs=[pl.BlockSpec((_EMBED_BLK, FEAT_DIM), lambda i: (i, 0)),
                   pl.BlockSpec((_EMBED_BLK, FEAT_DIM), lambda i: (i, 0))],
        out_shape=[jax.ShapeDtypeStruct((N_ITEMS, FEAT_DIM), jnp.float32),
                   jax.ShapeDtypeStruct((N_ITEMS, FEAT_DIM), jnp.float32)],
    )(image_feats, text_feats, W_img, W_txt, bias)


_E = 400000
_CHUNK = 64
_TCH = 392
_EPAD = 16 * _TCH * _CHUNK
_KBUF = 4

_U_SLAB = 15000
_I_SLAB = 10000
_SLAB_ALLOC = 15104
_TRASH = 15100
_FROWS = _SLAB_ALLOC // 16
_HALF = FEAT_DIM


def _spmm_body(y0, y1, srcu, srci, dstu, dsti, zeros, out, isrc, idst, gbuf,
               slab, gsem, ssem):
    c = lax.axis_index("c")
    sid = lax.axis_index("s")

    def run_phase(h, d, y_h, src_hbm, dst_hbm):
        pltpu.sync_copy(zeros.at[pl.ds(sid * _FROWS, _FROWS)],
                        slab.at[pl.ds(sid * _FROWS, _FROWS)])
        pltpu.sync_copy(src_hbm.at[sid], isrc)
        pltpu.sync_copy(dst_hbm.at[c, sid], idst)
        plsc.subcore_barrier()
        for k in range(_KBUF - 1):
            pltpu.async_copy(y_h.at[isrc.at[k]], gbuf.at[k], gsem.at[k])

        def step(j, carry):
            b = lax.bitwise_and(j, _KBUF - 1)
            pltpu.make_async_copy(y_h.at[isrc.at[j]], gbuf.at[b], gsem.at[b]).wait()
            pltpu.async_copy(gbuf.at[b], slab.at[idst.at[j]], ssem.at[b], add=True)

            @pl.when(j >= 1)
            def _():
                bp = lax.bitwise_and(j - 1, _KBUF - 1)
                pltpu.make_async_copy(gbuf.at[bp], slab.at[idst.at[j - 1]],
                                      ssem.at[bp]).wait()

            nxt = j + _KBUF - 1

            @pl.when(nxt < _TCH)
            def _():
                bn = lax.bitwise_and(nxt, _KBUF - 1)
                pltpu.async_copy(y_h.at[isrc.at[nxt]], gbuf.at[bn], gsem.at[bn])
            return carry

        lax.fori_loop(0, _TCH, step, 0)
        bl = (_TCH - 1) & (_KBUF - 1)
        pltpu.make_async_copy(gbuf.at[bl], slab.at[idst.at[_TCH - 1]],
                              ssem.at[bl]).wait()
        plsc.subcore_barrier()
        pltpu.sync_copy(slab.at[pl.ds(sid * _FROWS, _FROWS)],
                        out.at[h, d, c, pl.ds(sid * _FROWS, _FROWS)])
        plsc.subcore_barrier()

    for h, y_h in ((0, y0), (1, y1)):
        run_phase(h, 0, y_h, srcu, dstu)
        run_phase(h, 1, y_h, srci, dsti)


_sc_spmm = pl.kernel(
    _spmm_body,
    out_type=jax.ShapeDtypeStruct((2, 2, 2, _SLAB_ALLOC, _HALF), jnp.float32),
    mesh=plsc.VectorSubcoreMesh(core_axis_name="c", subcore_axis_name="s"),
    scratch_types=[
        pltpu.VMEM((_TCH, _CHUNK), jnp.int32),
        pltpu.VMEM((_TCH, _CHUNK), jnp.int32),
        pltpu.VMEM((_KBUF, _CHUNK, _HALF), jnp.float32),
        pltpu.VMEM_SHARED((_SLAB_ALLOC, _HALF), jnp.float32),
        pltpu.SemaphoreType.DMA((_KBUF,)),
        pltpu.SemaphoreType.DMA((_KBUF,)),
    ],
    compiler_params=pltpu.CompilerParams(use_tc_tiling_on_sc=False),
)


_DCH = 196
_DPAD = 32 * _DCH * 128
_DEG_TRASH = 51100
_HIST_WORDS = 51200


def _deg_body(rows_hbm, out, idxv, hist):
    c = lax.axis_index("c")
    sid = lax.axis_index("s")

    def zstep(i, carry):
        hist[pl.ds(i * 16, 16)] = jnp.zeros((16,), jnp.float32)
        return carry

    lax.fori_loop(0, _HIST_WORDS // 16, zstep, 0)
    pltpu.sync_copy(rows_hbm.at[c, sid], idxv)
    ones = jnp.ones((16,), jnp.float32)

    def step(ch, carry):
        for jj in range(8):
            idx16 = idxv[ch, pl.ds(jj * 16, 16)]
            plsc.addupdate_scatter(hist, [idx16], ones)
        return carry

    lax.fori_loop(0, _DCH, step, 0)
    pltpu.sync_copy(hist, out.at[c, sid])


_sc_degree = pl.kernel(
    _deg_body,
    out_type=jax.ShapeDtypeStruct((2, 16, _HIST_WORDS), jnp.float32),
    mesh=plsc.VectorSubcoreMesh(core_axis_name="c", subcore_axis_name="s"),
    scratch_types=[
        pltpu.VMEM((_DCH, 128), jnp.int32),
        pltpu.VMEM((_HIST_WORDS,), jnp.float32),
    ],
    compiler_params=pltpu.CompilerParams(needs_layout_passes=False,
                                         use_tc_tiling_on_sc=False),
)


def _pad_chunks(a, pad):
    return jnp.concatenate([a.astype(jnp.int32), pad]).reshape(16, _TCH, _CHUNK)


def _dst_lists(idx, half):
    spread = half + (jnp.arange(_E, dtype=jnp.int32) % (_SLAB_ALLOC - half))
    pad = _TRASH + (jnp.arange(_EPAD - _E, dtype=jnp.int32) % (_SLAB_ALLOC - _TRASH))
    lo = jnp.where(idx < half, idx, spread)
    hi = jnp.where(idx >= half, idx - half, spread)
    return jnp.stack([_pad_chunks(lo, pad), _pad_chunks(hi, pad)])


def kernel(user_idx, item_idx, image_feats, text_feats, image_preference,
           text_preference, W_img, b_img, W_txt, b_txt):
    emb0, emb1 = _dense_embed(image_feats, text_feats, W_img, b_img, W_txt, b_txt)
    x0 = jnp.concatenate([image_preference, emb0], axis=0)
    x1 = jnp.concatenate([text_preference, emb1], axis=0)

    srcpad = jnp.arange(_EPAD - _E, dtype=jnp.int32) % N_NODES
    srcu = _pad_chunks(item_idx + N_USERS, srcpad)
    srci = _pad_chunks(user_idx, srcpad)
    dstu = _dst_lists(user_idx, _U_SLAB)
    dsti = _dst_lists(item_idx, _I_SLAB)
    zeros = jnp.zeros((_SLAB_ALLOC, _HALF), jnp.float32)

    rows = jnp.concatenate([user_idx, item_idx + N_USERS]).astype(jnp.int32)
    rows = jnp.concatenate(
        [rows, jnp.full((_DPAD - 2 * _E,), _DEG_TRASH, jnp.int32)])
    rows = rows.reshape(2, 16, _DCH, 128)
    deg = _sc_degree(rows).sum(axis=(0, 1))[:N_NODES]
    d_inv = jnp.where(deg > 0, jax.lax.rsqrt(deg), 0.0)[:, None]

    def gather_s(o, h):
        return jnp.concatenate([o[h, 0, 0, :_U_SLAB], o[h, 0, 1, :_U_SLAB],
                                o[h, 1, 0, :_I_SLAB], o[h, 1, 1, :_I_SLAB]],
                               axis=0)

    for _ in range(N_LAYERS):
        o = _sc_spmm(x0 * d_inv, x1 * d_inv, srcu, srci, dstu, dsti, zeros)
        x0 = gather_s(o, 0) * d_inv + ALPHA * x0
        x1 = gather_s(o, 1) * d_inv + ALPHA * x1

    return (jnp.concatenate([x0[:N_USERS], x1[:N_USERS]], axis=1),
            jnp.concatenate([x0[N_USERS:], x1[N_USERS:]], axis=1))


# speedup vs baseline: 1.0177x; 1.0177x over previous
import functools

import jax
import jax.numpy as jnp
from jax import lax
from jax.experimental import pallas as pl
from jax.experimental.pallas import tpu as pltpu, tpu_sc as plsc

N_USERS = 30000
N_ITEMS = 20000
N_NODES = N_USERS + N_ITEMS
N_LAYERS = 2
ALPHA = 0.4
FEAT_DIM = 64


_EMBED_BLK = 1000


def _embed_body(img_ref, txt_ref, wi_ref, wt_ref, bias_ref, out_ref):
    ie = jnp.dot(img_ref[...], wi_ref[...], preferred_element_type=jnp.float32)
    te = jnp.dot(txt_ref[...], wt_ref[...], preferred_element_type=jnp.float32)
    e = jnp.concatenate([ie, te], axis=1) + bias_ref[...]
    ni = jnp.sqrt(jnp.sum(e[:, :FEAT_DIM] * e[:, :FEAT_DIM], axis=1, keepdims=True))
    nt = jnp.sqrt(jnp.sum(e[:, FEAT_DIM:] * e[:, FEAT_DIM:], axis=1, keepdims=True))
    n = jnp.concatenate([jnp.broadcast_to(ni, (ni.shape[0], FEAT_DIM)),
                         jnp.broadcast_to(nt, (nt.shape[0], FEAT_DIM))], axis=1)
    out_ref[...] = e / jnp.maximum(n, 1e-12)


def _dense_embed(image_feats, text_feats, W_img, b_img, W_txt, b_txt):
    bias = jnp.concatenate([b_img, b_txt]).reshape(1, 2 * FEAT_DIM)
    return pl.pallas_call(
        _embed_body,
        grid=(N_ITEMS // _EMBED_BLK,),
        in_specs=[
            pl.BlockSpec((_EMBED_BLK, image_feats.shape[1]), lambda i: (i, 0)),
            pl.BlockSpec((_EMBED_BLK, text_feats.shape[1]), lambda i: (i, 0)),
            pl.BlockSpec(W_img.shape, lambda i: (0, 0)),
            pl.BlockSpec(W_txt.shape, lambda i: (0, 0)),
            pl.BlockSpec((1, 2 * FEAT_DIM), lambda i: (0, 0)),
        ],
        out_specs=pl.BlockSpec((_EMBED_BLK, 2 * FEAT_DIM), lambda i: (i, 0)),
        out_shape=jax.ShapeDtypeStruct((N_ITEMS, 2 * FEAT_DIM), jnp.float32),
    )(image_feats, text_feats, W_img, W_txt, bias)


_E = 400000
_CHUNK = 64
_TCH = 392
_EPAD = 16 * _TCH * _CHUNK
_KBUF = 4

_U_SLAB = 15000
_I_SLAB = 10000
_SLAB_ALLOC = 15104
_TRASH = 15100
_FROWS = _SLAB_ALLOC // 16
_HALF = FEAT_DIM


def _spmm_body(y0, y1, srcu, srci, dstu, dsti, zeros, out, isrc, idst, gbuf,
               slab, gsem, ssem):
    c = lax.axis_index("c")
    sid = lax.axis_index("s")

    def run_phase(h, d, y_h, src_hbm, dst_hbm):
        pltpu.sync_copy(zeros.at[pl.ds(sid * _FROWS, _FROWS)],
                        slab.at[pl.ds(sid * _FROWS, _FROWS)])
        pltpu.sync_copy(src_hbm.at[sid], isrc)
        pltpu.sync_copy(dst_hbm.at[c, sid], idst)
        plsc.subcore_barrier()
        for k in range(_KBUF - 1):
            pltpu.async_copy(y_h.at[isrc.at[k]], gbuf.at[k], gsem.at[k])

        def step(j, carry):
            b = lax.bitwise_and(j, _KBUF - 1)
            pltpu.make_async_copy(y_h.at[isrc.at[j]], gbuf.at[b], gsem.at[b]).wait()
            pltpu.async_copy(gbuf.at[b], slab.at[idst.at[j]], ssem.at[b], add=True)

            @pl.when(j >= 1)
            def _():
                bp = lax.bitwise_and(j - 1, _KBUF - 1)
                pltpu.make_async_copy(gbuf.at[bp], slab.at[idst.at[j - 1]],
                                      ssem.at[bp]).wait()

            nxt = j + _KBUF - 1

            @pl.when(nxt < _TCH)
            def _():
                bn = lax.bitwise_and(nxt, _KBUF - 1)
                pltpu.async_copy(y_h.at[isrc.at[nxt]], gbuf.at[bn], gsem.at[bn])
            return carry

        lax.fori_loop(0, _TCH, step, 0)
        bl = (_TCH - 1) & (_KBUF - 1)
        pltpu.make_async_copy(gbuf.at[bl], slab.at[idst.at[_TCH - 1]],
                              ssem.at[bl]).wait()
        plsc.subcore_barrier()
        pltpu.sync_copy(slab.at[pl.ds(sid * _FROWS, _FROWS)],
                        out.at[h, d, c, pl.ds(sid * _FROWS, _FROWS)])
        plsc.subcore_barrier()

    for h, y_h in ((0, y0), (1, y1)):
        run_phase(h, 0, y_h, srcu, dstu)
        run_phase(h, 1, y_h, srci, dsti)


_sc_spmm = pl.kernel(
    _spmm_body,
    out_type=jax.ShapeDtypeStruct((2, 2, 2, _SLAB_ALLOC, _HALF), jnp.float32),
    mesh=plsc.VectorSubcoreMesh(core_axis_name="c", subcore_axis_name="s"),
    scratch_types=[
        pltpu.VMEM((_TCH, _CHUNK), jnp.int32),
        pltpu.VMEM((_TCH, _CHUNK), jnp.int32),
        pltpu.VMEM((_KBUF, _CHUNK, _HALF), jnp.float32),
        pltpu.VMEM_SHARED((_SLAB_ALLOC, _HALF), jnp.float32),
        pltpu.SemaphoreType.DMA((_KBUF,)),
        pltpu.SemaphoreType.DMA((_KBUF,)),
    ],
    compiler_params=pltpu.CompilerParams(use_tc_tiling_on_sc=False),
)


_DCH = 196
_DPAD = 32 * _DCH * 128
_DEG_TRASH = 51100
_HIST_WORDS = 51200


def _deg_body(rows_hbm, out, idxv, hist):
    c = lax.axis_index("c")
    sid = lax.axis_index("s")

    def zstep(i, carry):
        hist[pl.ds(i * 16, 16)] = jnp.zeros((16,), jnp.float32)
        return carry

    lax.fori_loop(0, _HIST_WORDS // 16, zstep, 0)
    pltpu.sync_copy(rows_hbm.at[c, sid], idxv)
    ones = jnp.ones((16,), jnp.float32)

    def step(ch, carry):
        for jj in range(8):
            idx16 = idxv[ch, pl.ds(jj * 16, 16)]
            plsc.addupdate_scatter(hist, [idx16], ones)
        return carry

    lax.fori_loop(0, _DCH, step, 0)
    pltpu.sync_copy(hist, out.at[c, sid])


_sc_degree = pl.kernel(
    _deg_body,
    out_type=jax.ShapeDtypeStruct((2, 16, _HIST_WORDS), jnp.float32),
    mesh=plsc.VectorSubcoreMesh(core_axis_name="c", subcore_axis_name="s"),
    scratch_types=[
        pltpu.VMEM((_DCH, 128), jnp.int32),
        pltpu.VMEM((_HIST_WORDS,), jnp.float32),
    ],
    compiler_params=pltpu.CompilerParams(needs_layout_passes=False,
                                         use_tc_tiling_on_sc=False),
)


def _pad_chunks(a, pad):
    return jnp.concatenate([a.astype(jnp.int32), pad]).reshape(16, _TCH, _CHUNK)


def _dst_lists(idx, half):
    spread = half + (jnp.arange(_E, dtype=jnp.int32) % (_SLAB_ALLOC - half))
    pad = _TRASH + (jnp.arange(_EPAD - _E, dtype=jnp.int32) % (_SLAB_ALLOC - _TRASH))
    lo = jnp.where(idx < half, idx, spread)
    hi = jnp.where(idx >= half, idx - half, spread)
    return jnp.stack([_pad_chunks(lo, pad), _pad_chunks(hi, pad)])


def kernel(user_idx, item_idx, image_feats, text_feats, image_preference,
           text_preference, W_img, b_img, W_txt, b_txt):
    items_emb = _dense_embed(image_feats, text_feats, W_img, b_img, W_txt, b_txt)
    users0 = jnp.concatenate([image_preference, text_preference], axis=1)
    x = jnp.concatenate([users0, items_emb], axis=0)

    srcpad = jnp.arange(_EPAD - _E, dtype=jnp.int32) % N_NODES
    srcu = _pad_chunks(item_idx + N_USERS, srcpad)
    srci = _pad_chunks(user_idx, srcpad)
    dstu = _dst_lists(user_idx, _U_SLAB)
    dsti = _dst_lists(item_idx, _I_SLAB)
    zeros = jnp.zeros((_SLAB_ALLOC, _HALF), jnp.float32)

    rows = jnp.concatenate([user_idx, item_idx + N_USERS]).astype(jnp.int32)
    rows = jnp.concatenate(
        [rows, jnp.full((_DPAD - 2 * _E,), _DEG_TRASH, jnp.int32)])
    rows = rows.reshape(2, 16, _DCH, 128)
    deg = _sc_degree(rows).sum(axis=(0, 1))[:N_NODES]
    d_inv = jnp.where(deg > 0, jax.lax.rsqrt(deg), 0.0)[:, None]

    for _ in range(N_LAYERS):
        y = x * d_inv
        o = _sc_spmm(y[:, :_HALF], y[:, _HALF:], srcu, srci, dstu, dsti, zeros)
        s = jnp.concatenate(
            [jnp.concatenate([o[h, 0, 0, :_U_SLAB], o[h, 0, 1, :_U_SLAB],
                              o[h, 1, 0, :_I_SLAB], o[h, 1, 1, :_I_SLAB]],
                             axis=0) for h in (0, 1)], axis=1)
        x = s * d_inv + ALPHA * x

    return (x[:N_USERS], x[N_USERS:])
